# Initial kernel scaffold; baseline (speedup 1.0000x reference)
import functools

import jax
import jax.numpy as jnp
from jax import lax
from jax.experimental import pallas as pl
from jax.experimental.pallas import tpu as pltpu
from jax.experimental.pallas import tpu_sc as plsc

N_NODES = 10000
D_IN = 128
D_OUT = 128

NC = 2
NS = 16
NW = NC * NS

CHUNK = 128
AUG = 144
NPAD = 10240
ROWS_PER_TILE = NPAD // NS


def _sc_aggregate(n_chunks):
    mesh = plsc.VectorSubcoreMesh(core_axis_name="c", subcore_axis_name="s")

    @functools.partial(
        pl.kernel,
        out_type=jax.ShapeDtypeStruct((NC, NPAD, AUG), jnp.float32),
        mesh=mesh,
        scratch_types=[
            pltpu.VMEM((n_chunks, CHUNK), jnp.int32),
            pltpu.VMEM((n_chunks, CHUNK), jnp.int32),
            pltpu.VMEM((CHUNK, AUG), jnp.float32),
            pltpu.VMEM_SHARED((NPAD, AUG), jnp.float32),
            pltpu.SemaphoreType.DMA,
        ],
    )
    def sc_agg(h_aug, src3, dst3, zeros, out, src_v, dst_v, rows_v, acc, sem):
        cid = lax.axis_index("c")
        sid = lax.axis_index("s")
        wid = cid * NS + sid
        r0 = sid * ROWS_PER_TILE

        pltpu.sync_copy(zeros.at[pl.ds(r0, ROWS_PER_TILE)],
                        acc.at[pl.ds(r0, ROWS_PER_TILE)])
        pltpu.sync_copy(src3.at[wid], src_v)
        pltpu.sync_copy(dst3.at[wid], dst_v)
        plsc.subcore_barrier()

        def body(c, carry):
            pltpu.async_copy(h_aug.at[src_v.at[c]], rows_v, sem).wait()
            pltpu.sync_copy(rows_v, acc.at[dst_v.at[c]], add=True)
            return carry

        lax.fori_loop(0, n_chunks, body, 0)

        plsc.subcore_barrier()
        pltpu.sync_copy(acc.at[pl.ds(r0, ROWS_PER_TILE)],
                        out.at[cid, pl.ds(r0, ROWS_PER_TILE)])

    return sc_agg


def _tc_combine(h_blk, parts_blk, w_blk, b_blk, out_blk):
    p = parts_blk[0] + parts_blk[1]
    s = p[:, :D_IN]
    deg = p[:, D_IN:D_IN + 1]
    agg = s / jnp.maximum(deg, 1.0)
    out_blk[...] = (
        jnp.dot(h_blk[...], w_blk[:D_IN], preferred_element_type=jnp.float32)
        + jnp.dot(agg, w_blk[D_IN:], preferred_element_type=jnp.float32)
        + b_blk[...]
    )


def kernel(h, edge_index, W, b):
    src = edge_index[0].astype(jnp.int32)
    dst = edge_index[1].astype(jnp.int32)
    n_edges = src.shape[0]

    per_tile = -(-n_edges // (NW * CHUNK)) * CHUNK
    n_chunks = per_tile // CHUNK
    e_pad = NW * per_tile
    src = jnp.concatenate([src, jnp.zeros((e_pad - n_edges,), jnp.int32)])
    dst = jnp.concatenate(
        [dst, jnp.full((e_pad - n_edges,), N_NODES, jnp.int32)])
    src3 = src.reshape(NW, n_chunks, CHUNK)
    dst3 = dst.reshape(NW, n_chunks, CHUNK)

    h_aug = jnp.concatenate(
        [h, jnp.ones((N_NODES, 1), h.dtype),
         jnp.zeros((N_NODES, AUG - D_IN - 1), h.dtype)], axis=1)
    zeros = jnp.zeros((NPAD, AUG), jnp.float32)

    parts = _sc_aggregate(n_chunks)(h_aug, src3, dst3, zeros)

    blk = 1000
    grid = N_NODES // blk
    out = pl.pallas_call(
        _tc_combine,
        grid=(grid,),
        in_specs=[
            pl.BlockSpec((blk, D_IN), lambda i: (i, 0)),
            pl.BlockSpec((NC, blk, AUG), lambda i: (0, i, 0)),
            pl.BlockSpec((2 * D_IN, D_OUT), lambda i: (0, 0)),
            pl.BlockSpec((1, D_OUT), lambda i: (0, 0)),
        ],
        out_specs=pl.BlockSpec((blk, D_OUT), lambda i: (i, 0)),
        out_shape=jax.ShapeDtypeStruct((N_NODES, D_OUT), jnp.float32),
    )(h, parts, W, b.reshape(1, D_OUT))
    return out


# speedup vs baseline: 5.6003x; 5.6003x over previous
import functools

import jax
import jax.numpy as jnp
from jax import lax
from jax.experimental import pallas as pl
from jax.experimental.pallas import tpu as pltpu
from jax.experimental.pallas import tpu_sc as plsc

N_NODES = 10000
D_IN = 128
D_OUT = 128

NC = 2
NS = 16
NW = NC * NS

CHUNK = 128
AUG = 144
NPAD = 10240
ROWS_PER_TILE = NPAD // NS


def _sc_aggregate(n_chunks):
    mesh = plsc.VectorSubcoreMesh(core_axis_name="c", subcore_axis_name="s")

    @functools.partial(
        pl.kernel,
        out_type=jax.ShapeDtypeStruct((NC, NPAD, AUG), jnp.float32),
        mesh=mesh,
        compiler_params=pltpu.CompilerParams(use_tc_tiling_on_sc=False),
        scratch_types=[
            pltpu.VMEM((n_chunks, CHUNK), jnp.int32),
            pltpu.VMEM((n_chunks, CHUNK), jnp.int32),
            pltpu.VMEM((CHUNK, AUG), jnp.float32),
            pltpu.VMEM_SHARED((NPAD, AUG), jnp.float32),
            pltpu.SemaphoreType.DMA,
        ],
    )
    def sc_agg(h_aug, src3, dst3, zeros, out, src_v, dst_v, rows_v, acc, sem):
        cid = lax.axis_index("c")
        sid = lax.axis_index("s")
        wid = cid * NS + sid
        r0 = sid * ROWS_PER_TILE

        pltpu.sync_copy(zeros.at[pl.ds(r0, ROWS_PER_TILE)],
                        acc.at[pl.ds(r0, ROWS_PER_TILE)])
        pltpu.sync_copy(src3.at[wid], src_v)
        pltpu.sync_copy(dst3.at[wid], dst_v)
        plsc.subcore_barrier()

        def body(c, carry):
            pltpu.async_copy(h_aug.at[src_v.at[c]], rows_v, sem).wait()
            pltpu.sync_copy(rows_v, acc.at[dst_v.at[c]], add=True)
            return carry

        lax.fori_loop(0, n_chunks, body, 0)

        plsc.subcore_barrier()
        pltpu.sync_copy(acc.at[pl.ds(r0, ROWS_PER_TILE)],
                        out.at[cid, pl.ds(r0, ROWS_PER_TILE)])

    return sc_agg


def _tc_combine(h_blk, parts_blk, w_blk, b_blk, out_blk):
    p = parts_blk[0] + parts_blk[1]
    s = p[:, :D_IN]
    deg = p[:, D_IN:D_IN + 1]
    agg = s / jnp.maximum(deg, 1.0)
    out_blk[...] = (
        jnp.dot(h_blk[...], w_blk[:D_IN], preferred_element_type=jnp.float32)
        + jnp.dot(agg, w_blk[D_IN:], preferred_element_type=jnp.float32)
        + b_blk[...]
    )


def kernel(h, edge_index, W, b):
    src = edge_index[0].astype(jnp.int32)
    dst = edge_index[1].astype(jnp.int32)
    n_edges = src.shape[0]

    per_tile = -(-n_edges // (NW * CHUNK)) * CHUNK
    n_chunks = per_tile // CHUNK
    e_pad = NW * per_tile
    src = jnp.concatenate([src, jnp.zeros((e_pad - n_edges,), jnp.int32)])
    dst = jnp.concatenate(
        [dst, jnp.full((e_pad - n_edges,), N_NODES, jnp.int32)])
    src3 = src.reshape(NW, n_chunks, CHUNK)
    dst3 = dst.reshape(NW, n_chunks, CHUNK)

    h_aug = jnp.concatenate(
        [h, jnp.ones((N_NODES, 1), h.dtype),
         jnp.zeros((N_NODES, AUG - D_IN - 1), h.dtype)], axis=1)
    zeros = jnp.zeros((NPAD, AUG), jnp.float32)

    parts = _sc_aggregate(n_chunks)(h_aug, src3, dst3, zeros)

    blk = 1000
    grid = N_NODES // blk
    out = pl.pallas_call(
        _tc_combine,
        grid=(grid,),
        in_specs=[
            pl.BlockSpec((blk, D_IN), lambda i: (i, 0)),
            pl.BlockSpec((NC, blk, AUG), lambda i: (0, i, 0)),
            pl.BlockSpec((2 * D_IN, D_OUT), lambda i: (0, 0)),
            pl.BlockSpec((1, D_OUT), lambda i: (0, 0)),
        ],
        out_specs=pl.BlockSpec((blk, D_OUT), lambda i: (i, 0)),
        out_shape=jax.ShapeDtypeStruct((N_NODES, D_OUT), jnp.float32),
    )(h, parts, W, b.reshape(1, D_OUT))
    return out
